# baseline (speedup 1.0000x reference)
import jax
import jax.numpy as jnp
from jax import lax
from jax.experimental import pallas as pl
from jax.experimental.pallas import tpu as pltpu


def _conv1_kernel(w_ref, b_ref, xr_ref, p_ref, o_ref, xs_ref, acc_ref):
    n_co = 6
    phases = ((0, 0), (0, 1), (1, 0), (1, 1))

    for ci in range(3):
        for by in range(2):
            for s in range(6):
                xs_ref[ci, by, s] = jnp.dot(
                    xr_ref[0, ci, by], p_ref[s],
                    preferred_element_type=jnp.float32)

    acc_ref[...] = jnp.zeros_like(acc_ref)

    def tap_body(g, carry):
        for st in range(5):
            p0 = 24 * st
            sls = []
            wss = []
            for k in range(3):
                cidy = g * 3 + k
                ci = cidy // 5
                dy = cidy % 5
                wss.append([[w_ref[co, cidy * 5 + dx] for dx in range(5)]
                            for co in range(n_co)])
                sls.append([[xs_ref[ci, (ry + dy) % 2, s6,
                                    pl.ds(p0 + (ry + dy) // 2, 24), :]
                             for s6 in range(6)] for ry in range(2)])
            for ph, (ry, rx) in enumerate(phases):
                for co in range(n_co):
                    a = acc_ref[ph * n_co + co, pl.ds(p0, 24), :]
                    for k in range(3):
                        for dx in range(5):
                            a = a + wss[k][co][dx] * sls[k][ry][rx + dx]
                    acc_ref[ph * n_co + co, pl.ds(p0, 24), :] = a
        return carry

    lax.fori_loop(0, 5, tap_body, 0)

    for co in range(n_co):
        m = jnp.maximum(jnp.maximum(acc_ref[co], acc_ref[n_co + co]),
                        jnp.maximum(acc_ref[2 * n_co + co],
                                    acc_ref[3 * n_co + co]))
        o_ref[0, co] = jnp.maximum(m[:, :120] + b_ref[co, 0], 0.0)


def _conv1(x, w, b):
    n = x.shape[0]
    xr = x.reshape(n, 3, 122, 2, 244).transpose(0, 1, 3, 2, 4)
    c = jnp.arange(128)
    s = jnp.arange(6)
    src = 2 * c[None, :] + s[:, None]
    p1 = ((jnp.arange(244)[None, :, None] == src[:, None, :]) &
          (c[None, None, :] < 120)).astype(jnp.float32)
    w2 = w.reshape(6, 75)
    b2 = b.reshape(6, 1)
    return pl.pallas_call(
        _conv1_kernel,
        out_shape=jax.ShapeDtypeStruct((n, 6, 120, 120), jnp.float32),
        grid=(n,),
        in_specs=[
            pl.BlockSpec(memory_space=pltpu.SMEM),
            pl.BlockSpec(memory_space=pltpu.SMEM),
            pl.BlockSpec((1, 3, 2, 122, 244), lambda i: (i, 0, 0, 0, 0)),
            pl.BlockSpec((6, 244, 128), lambda i: (0, 0, 0)),
        ],
        out_specs=pl.BlockSpec((1, 6, 120, 120), lambda i: (i, 0, 0, 0)),
        scratch_shapes=[pltpu.VMEM((3, 2, 6, 122, 128), jnp.float32),
                        pltpu.VMEM((24, 120, 128), jnp.float32)],
        compiler_params=pltpu.CompilerParams(
            dimension_semantics=("parallel",)),
    )(w2, b2, xr, p1)


def _conv2_kernel(w_ref, b_ref, xr_ref, p_ref, o_ref, xp_ref, acc_ref):
    n_co = 16

    for ci in range(6):
        for by in range(2):
            for dx in range(5):
                xp_ref[ci, by, dx] = jnp.dot(
                    xr_ref[0, ci, by], p_ref[dx],
                    preferred_element_type=jnp.float32)

    acc_ref[...] = jnp.zeros_like(acc_ref)

    def tap_body(g, carry):
        for st in range(4):
            p0 = 16 * st
            sls = []
            wss = []
            for k in range(3):
                cidy = g * 3 + k
                ci = cidy // 5
                dy = cidy % 5
                wss.append([[w_ref[co, cidy * 5 + dx] for dx in range(5)]
                            for co in range(n_co)])
                sls.append([[xp_ref[ci, (ry + dy) % 2, dx,
                                    pl.ds(p0 + (ry + dy) // 2, 16), :]
                             for dx in range(5)] for ry in range(2)])
            for ry in range(2):
                for co in range(n_co):
                    a = acc_ref[ry * n_co + co, pl.ds(p0, 16), :]
                    for k in range(3):
                        for dx in range(5):
                            a = a + wss[k][co][dx] * sls[k][ry][dx]
                    acc_ref[ry * n_co + co, pl.ds(p0, 16), :] = a
        return carry

    lax.fori_loop(0, 10, tap_body, 0)

    for co in range(n_co):
        m = jnp.maximum(acc_ref[co], acc_ref[n_co + co])
        m = jnp.maximum(m[:, :58], m[:, 58:116])
        o_ref[0, co] = jnp.maximum(m[:58, :] + b_ref[co, 0], 0.0)


def _conv2(x1, w, b):
    n = x1.shape[0]
    xr = x1.reshape(n, 6, 60, 2, 120).transpose(0, 1, 3, 2, 4)
    xr = jnp.pad(xr, ((0, 0), (0, 0), (0, 0), (0, 12), (0, 0)))
    c = jnp.arange(128)
    dxb = jnp.arange(5)
    src0 = 2 * c[None, :] + dxb[:, None]
    src1 = 2 * (c[None, :] - 58) + 1 + dxb[:, None]
    src = jnp.where(c[None, :] < 58, src0, src1)
    p2 = ((jnp.arange(120)[None, :, None] == src[:, None, :]) &
          (c[None, None, :] < 116)).astype(jnp.float32)
    w2 = w.reshape(16, 150)
    b2 = b.reshape(16, 1)
    return pl.pallas_call(
        _conv2_kernel,
        out_shape=jax.ShapeDtypeStruct((n, 16, 58, 58), jnp.float32),
        grid=(n,),
        in_specs=[
            pl.BlockSpec(memory_space=pltpu.SMEM),
            pl.BlockSpec(memory_space=pltpu.SMEM),
            pl.BlockSpec((1, 6, 2, 72, 120), lambda i: (i, 0, 0, 0, 0)),
            pl.BlockSpec((5, 120, 128), lambda i: (0, 0, 0)),
        ],
        out_specs=pl.BlockSpec((1, 16, 58, 58), lambda i: (i, 0, 0, 0)),
        scratch_shapes=[pltpu.VMEM((6, 2, 5, 72, 128), jnp.float32),
                        pltpu.VMEM((32, 64, 128), jnp.float32)],
        compiler_params=pltpu.CompilerParams(
            dimension_semantics=("parallel",)),
    )(w2, b2, xr, p2)


FC1_TK = 8192


def _fc1_kernel(x_ref, w_ref, b_ref, o_ref, acc_ref):
    k = pl.program_id(1)

    @pl.when(k == 0)
    def _():
        acc_ref[...] = jnp.zeros_like(acc_ref)

    acc_ref[...] += jnp.dot(x_ref[...], w_ref[...],
                            preferred_element_type=jnp.float32)

    @pl.when(k == pl.num_programs(1) - 1)
    def _():
        o_ref[...] = jnp.maximum(acc_ref[...] + b_ref[...], 0.0)


def _fc23_kernel(h_ref, w2_ref, b2_ref, w3_ref, b3_ref, o_ref):
    h2 = jnp.maximum(
        jnp.dot(h_ref[...], w2_ref[...], preferred_element_type=jnp.float32)
        + b2_ref[...], 0.0)
    o_ref[...] = (jnp.dot(h2, w3_ref[...], preferred_element_type=jnp.float32)
                  + b3_ref[...])


def _fc_head(x2, w1, b1, w2, b2, w3, b3):
    m = x2.shape[0]
    kp = w1.shape[0]
    xp = jnp.pad(x2, ((0, 0), (0, kp - x2.shape[1])))
    nk = kp // FC1_TK
    h = pl.pallas_call(
        _fc1_kernel,
        out_shape=jax.ShapeDtypeStruct((m, 256), jnp.float32),
        grid=(2, nk),
        in_specs=[
            pl.BlockSpec((m, FC1_TK), lambda nh, k: (0, k)),
            pl.BlockSpec((FC1_TK, 128), lambda nh, k: (k, nh)),
            pl.BlockSpec((1, 128), lambda nh, k: (0, nh)),
        ],
        out_specs=pl.BlockSpec((m, 128), lambda nh, k: (0, nh)),
        scratch_shapes=[pltpu.VMEM((m, 128), jnp.float32)],
        compiler_params=pltpu.CompilerParams(
            dimension_semantics=("parallel", "arbitrary")),
    )(xp, w1, b1.reshape(1, 256))

    b2p = jnp.pad(b2, (0, 128 - b2.shape[0])).reshape(1, 128)
    b3p = jnp.pad(b3, (0, 128 - b3.shape[0])).reshape(1, 128)
    out = pl.pallas_call(
        _fc23_kernel,
        out_shape=jax.ShapeDtypeStruct((m, 128), jnp.float32),
    )(h, w2, b2p, w3, b3p)
    return out[:, :10]


def kernel(conv1_w, conv1_b, conv2_w, conv2_b, fc1_w_t, fc1_b,
           fc2_w_t, fc2_b, fc3_w_t, fc3_b, x):
    x1 = _conv1(x, conv1_w, conv1_b)
    x2 = _conv2(x1, conv2_w, conv2_b)
    n = x2.shape[0]
    flat = x2.reshape(n, 16 * 58 * 58)
    return _fc_head(flat, fc1_w_t, fc1_b, fc2_w_t, fc2_b, fc3_w_t, fc3_b)


# speedup vs baseline: 1.2071x; 1.2071x over previous
import jax
import jax.numpy as jnp
from jax import lax
from jax.experimental import pallas as pl
from jax.experimental.pallas import tpu as pltpu


def _conv1_kernel(w_ref, b_ref, xr_ref, p_ref, o_ref, xs_ref, acc_ref):
    n_co = 6
    phases = ((0, 0), (0, 1), (1, 0), (1, 1))

    for ci in range(3):
        for by in range(2):
            for s in range(6):
                xs_ref[ci, by, s] = jnp.dot(
                    xr_ref[0, ci, by], p_ref[s],
                    preferred_element_type=jnp.float32)

    acc_ref[...] = jnp.zeros_like(acc_ref)

    def tap_body(cidy, carry):
        ci = cidy // 5
        dy = cidy % 5
        ws = [[w_ref[co, cidy * 5 + dx] for dx in range(5)]
              for co in range(n_co)]
        bys = []
        ays = []
        for ry in range(2):
            bys.append((ry + dy) % 2)
            ays.append((ry + dy) // 2)
        for st in range(5):
            p0 = 24 * st
            sl = [[xs_ref[ci, bys[ry], s6, pl.ds(p0 + ays[ry], 24), :]
                   for s6 in range(6)] for ry in range(2)]
            for ph, (ry, rx) in enumerate(phases):
                for co in range(n_co):
                    a = acc_ref[ph * n_co + co, pl.ds(p0, 24), :]
                    for dx in range(5):
                        a = a + ws[co][dx] * sl[ry][rx + dx]
                    acc_ref[ph * n_co + co, pl.ds(p0, 24), :] = a
        return carry

    lax.fori_loop(0, 15, tap_body, 0)

    for co in range(n_co):
        m = jnp.maximum(jnp.maximum(acc_ref[co], acc_ref[n_co + co]),
                        jnp.maximum(acc_ref[2 * n_co + co],
                                    acc_ref[3 * n_co + co]))
        o_ref[0, co] = jnp.maximum(m[:, :120] + b_ref[co, 0], 0.0)


def _conv1(x, w, b):
    n = x.shape[0]
    xr = x.reshape(n, 3, 122, 2, 244).transpose(0, 1, 3, 2, 4)
    c = jnp.arange(128)
    s = jnp.arange(6)
    src = 2 * c[None, :] + s[:, None]
    p1 = ((jnp.arange(244)[None, :, None] == src[:, None, :]) &
          (c[None, None, :] < 120)).astype(jnp.float32)
    w2 = w.reshape(6, 75)
    b2 = b.reshape(6, 1)
    return pl.pallas_call(
        _conv1_kernel,
        out_shape=jax.ShapeDtypeStruct((n, 6, 120, 120), jnp.float32),
        grid=(n,),
        in_specs=[
            pl.BlockSpec(memory_space=pltpu.SMEM),
            pl.BlockSpec(memory_space=pltpu.SMEM),
            pl.BlockSpec((1, 3, 2, 122, 244), lambda i: (i, 0, 0, 0, 0)),
            pl.BlockSpec((6, 244, 128), lambda i: (0, 0, 0)),
        ],
        out_specs=pl.BlockSpec((1, 6, 120, 120), lambda i: (i, 0, 0, 0)),
        scratch_shapes=[pltpu.VMEM((3, 2, 6, 122, 128), jnp.float32),
                        pltpu.VMEM((24, 120, 128), jnp.float32)],
        compiler_params=pltpu.CompilerParams(
            dimension_semantics=("parallel",)),
    )(w2, b2, xr, p1)


def _conv2_kernel(w_ref, b_ref, xr_ref, p_ref, o_ref, xp_ref, acc_ref):
    n_co = 16

    for ci in range(6):
        for by in range(2):
            for dx in range(5):
                xp_ref[ci, by, dx] = jnp.dot(
                    xr_ref[0, ci, by], p_ref[dx],
                    preferred_element_type=jnp.float32)

    acc_ref[...] = jnp.zeros_like(acc_ref)

    def tap_body(cidy, carry):
        ci = cidy // 5
        dy = cidy % 5
        ws = [[w_ref[co, cidy * 5 + dx] for dx in range(5)]
              for co in range(n_co)]
        for st in range(4):
            p0 = 16 * st
            sl = []
            for ry in range(2):
                by = (ry + dy) % 2
                ay = (ry + dy) // 2
                sl.append([xp_ref[ci, by, dx, pl.ds(p0 + ay, 16), :]
                           for dx in range(5)])
            for ry in range(2):
                for co in range(n_co):
                    a = acc_ref[ry * n_co + co, pl.ds(p0, 16), :]
                    for dx in range(5):
                        a = a + ws[co][dx] * sl[ry][dx]
                    acc_ref[ry * n_co + co, pl.ds(p0, 16), :] = a
        return carry

    lax.fori_loop(0, 30, tap_body, 0)

    for co in range(n_co):
        m = jnp.maximum(acc_ref[co], acc_ref[n_co + co])
        m = jnp.maximum(m[:, :58], m[:, 58:116])
        o_ref[0, co] = jnp.maximum(m[:58, :] + b_ref[co, 0], 0.0)


def _conv2(x1, w, b):
    n = x1.shape[0]
    xr = x1.reshape(n, 6, 60, 2, 120).transpose(0, 1, 3, 2, 4)
    xr = jnp.pad(xr, ((0, 0), (0, 0), (0, 0), (0, 12), (0, 0)))
    c = jnp.arange(128)
    dxb = jnp.arange(5)
    src0 = 2 * c[None, :] + dxb[:, None]
    src1 = 2 * (c[None, :] - 58) + 1 + dxb[:, None]
    src = jnp.where(c[None, :] < 58, src0, src1)
    p2 = ((jnp.arange(120)[None, :, None] == src[:, None, :]) &
          (c[None, None, :] < 116)).astype(jnp.float32)
    w2 = w.reshape(16, 150)
    b2 = b.reshape(16, 1)
    return pl.pallas_call(
        _conv2_kernel,
        out_shape=jax.ShapeDtypeStruct((n, 16, 58, 58), jnp.float32),
        grid=(n,),
        in_specs=[
            pl.BlockSpec(memory_space=pltpu.SMEM),
            pl.BlockSpec(memory_space=pltpu.SMEM),
            pl.BlockSpec((1, 6, 2, 72, 120), lambda i: (i, 0, 0, 0, 0)),
            pl.BlockSpec((5, 120, 128), lambda i: (0, 0, 0)),
        ],
        out_specs=pl.BlockSpec((1, 16, 58, 58), lambda i: (i, 0, 0, 0)),
        scratch_shapes=[pltpu.VMEM((6, 2, 5, 72, 128), jnp.float32),
                        pltpu.VMEM((32, 64, 128), jnp.float32)],
        compiler_params=pltpu.CompilerParams(
            dimension_semantics=("parallel",)),
    )(w2, b2, xr, p2)


FC1_TK = 8192


def _fc1_kernel(x_ref, w_ref, b_ref, o_ref, acc_ref):
    k = pl.program_id(1)

    @pl.when(k == 0)
    def _():
        acc_ref[...] = jnp.zeros_like(acc_ref)

    acc_ref[...] += jnp.dot(x_ref[...], w_ref[...],
                            preferred_element_type=jnp.float32)

    @pl.when(k == pl.num_programs(1) - 1)
    def _():
        o_ref[...] = jnp.maximum(acc_ref[...] + b_ref[...], 0.0)


def _fc23_kernel(h_ref, w2_ref, b2_ref, w3_ref, b3_ref, o_ref):
    h2 = jnp.maximum(
        jnp.dot(h_ref[...], w2_ref[...], preferred_element_type=jnp.float32)
        + b2_ref[...], 0.0)
    o_ref[...] = (jnp.dot(h2, w3_ref[...], preferred_element_type=jnp.float32)
                  + b3_ref[...])


def _fc_head(x2, w1, b1, w2, b2, w3, b3):
    m = x2.shape[0]
    kp = w1.shape[0]
    xp = jnp.pad(x2, ((0, 0), (0, kp - x2.shape[1])))
    nk = kp // FC1_TK
    h = pl.pallas_call(
        _fc1_kernel,
        out_shape=jax.ShapeDtypeStruct((m, 256), jnp.float32),
        grid=(2, nk),
        in_specs=[
            pl.BlockSpec((m, FC1_TK), lambda nh, k: (0, k)),
            pl.BlockSpec((FC1_TK, 128), lambda nh, k: (k, nh)),
            pl.BlockSpec((1, 128), lambda nh, k: (0, nh)),
        ],
        out_specs=pl.BlockSpec((m, 128), lambda nh, k: (0, nh)),
        scratch_shapes=[pltpu.VMEM((m, 128), jnp.float32)],
        compiler_params=pltpu.CompilerParams(
            dimension_semantics=("parallel", "arbitrary")),
    )(xp, w1, b1.reshape(1, 256))

    b2p = jnp.pad(b2, (0, 128 - b2.shape[0])).reshape(1, 128)
    b3p = jnp.pad(b3, (0, 128 - b3.shape[0])).reshape(1, 128)
    out = pl.pallas_call(
        _fc23_kernel,
        out_shape=jax.ShapeDtypeStruct((m, 128), jnp.float32),
    )(h, w2, b2p, w3, b3p)
    return out[:, :10]


def kernel(conv1_w, conv1_b, conv2_w, conv2_b, fc1_w_t, fc1_b,
           fc2_w_t, fc2_b, fc3_w_t, fc3_b, x):
    x1 = _conv1(x, conv1_w, conv1_b)
    x2 = _conv2(x1, conv2_w, conv2_b)
    n = x2.shape[0]
    flat = x2.reshape(n, 16 * 58 * 58)
    return _fc_head(flat, fc1_w_t, fc1_b, fc2_w_t, fc2_b, fc3_w_t, fc3_b)
